# Initial kernel scaffold; baseline (speedup 1.0000x reference)
import jax
import jax.numpy as jnp
from jax.experimental import pallas as pl


def kernel(x, edge_index, batch, W1_0, b1_0, W2_0, b2_0, eps_0, W1_1, b1_1, W2_1, b2_1, eps_1, W1_2, b1_2, W2_2, b2_2, eps_2, fcW, fcb):
    raise NotImplementedError("write your pallas kernel here")


# speedup vs baseline: 4.2944x; 4.2944x over previous
import functools

import jax
import jax.numpy as jnp
from jax import lax
from jax.experimental import pallas as pl
from jax.experimental.pallas import tpu as pltpu
from jax.experimental.pallas import tpu_sc as plsc

N = 10000
E = 320000
D = 128
G = 64

NC = 2
NS = 16
NW = NC * NS
CHUNK = 128
K = -(-E // (NW * CHUNK))
E_PAD = NW * K * CHUNK
N_PAD = 10112
ZR = N_PAD // NS
BLK = 2528
TC_GRID = N_PAD // BLK


def _segment_sum_sc(x_pad, src3, dst3, zrows):
    mesh = plsc.VectorSubcoreMesh(core_axis_name="c", subcore_axis_name="s")

    @functools.partial(
        pl.kernel,
        mesh=mesh,
        out_type=jax.ShapeDtypeStruct((NC, N_PAD, D), jnp.float32),
        scratch_types=[
            pltpu.VMEM((K, CHUNK), jnp.int32),
            pltpu.VMEM((K, CHUNK), jnp.int32),
            pltpu.VMEM((CHUNK, D), jnp.float32),
            pltpu.VMEM_SHARED((N_PAD, D), jnp.float32),
            pltpu.SemaphoreType.DMA,
        ],
    )
    def seg_sum(x_hbm, src_hbm, dst_hbm, z_hbm, out_hbm,
                src_v, dst_v, rows_v, agg_sh, sem):
        c = lax.axis_index("c")
        s = lax.axis_index("s")
        wid = c * NS + s
        pltpu.sync_copy(src_hbm.at[wid], src_v)
        pltpu.sync_copy(dst_hbm.at[wid], dst_v)
        pltpu.sync_copy(z_hbm, agg_sh.at[pl.ds(s * ZR, ZR)])
        plsc.subcore_barrier()

        def body(j, carry):
            pltpu.async_copy(x_hbm.at[src_v.at[j]], rows_v, sem).wait()
            pltpu.sync_copy(rows_v, agg_sh.at[dst_v.at[j]], add=True)
            return carry

        lax.fori_loop(0, K, body, 0)
        plsc.subcore_barrier()
        pltpu.sync_copy(agg_sh.at[pl.ds(s * ZR, ZR)],
                        out_hbm.at[c, pl.ds(s * ZR, ZR)])

    return seg_sum(x_pad, src3, dst3, zrows)


def _mlp_tc(x, a0, a1, W1, b1, W2, b2, eps, outer_relu):
    def body(eps_ref, x_ref, a0_ref, a1_ref, w1_ref, b1_ref, w2_ref, b2_ref,
             o_ref):
        h = (1.0 + eps_ref[0]) * x_ref[...] + a0_ref[...] + a1_ref[...]
        h = jnp.maximum(
            jnp.dot(h, w1_ref[...], preferred_element_type=jnp.float32)
            + b1_ref[...], 0.0)
        h = jnp.dot(h, w2_ref[...], preferred_element_type=jnp.float32) \
            + b2_ref[...]
        if outer_relu:
            h = jnp.maximum(h, 0.0)
        o_ref[...] = h

    row_spec = pl.BlockSpec((BLK, D), lambda i: (i, 0))
    full_spec = pl.BlockSpec((D, D), lambda i: (0, 0))
    bias_spec = pl.BlockSpec((1, D), lambda i: (0, 0))
    return pl.pallas_call(
        body,
        grid=(TC_GRID,),
        in_specs=[
            pl.BlockSpec(memory_space=pltpu.SMEM),
            row_spec, row_spec, row_spec,
            full_spec, bias_spec, full_spec, bias_spec,
        ],
        out_specs=row_spec,
        out_shape=jax.ShapeDtypeStruct((N_PAD, D), jnp.float32),
    )(eps.reshape(1), x, a0, a1, W1, b1.reshape(1, D), W2, b2.reshape(1, D))


def _mlp_pool_tc(x, a0, a1, W1, b1, W2, b2, eps, batch2d, fcW, fcb):
    def body(eps_ref, fcb_ref, x_ref, a0_ref, a1_ref, w1_ref, b1_ref, w2_ref,
             b2_ref, batch_ref, fcw_ref, o_ref, pooled_s, counts_s):
        i = pl.program_id(0)

        @pl.when(i == 0)
        def _():
            pooled_s[...] = jnp.zeros_like(pooled_s)
            counts_s[...] = jnp.zeros_like(counts_s)

        h = (1.0 + eps_ref[0]) * x_ref[...] + a0_ref[...] + a1_ref[...]
        h = jnp.maximum(
            jnp.dot(h, w1_ref[...], preferred_element_type=jnp.float32)
            + b1_ref[...], 0.0)
        h = jnp.dot(h, w2_ref[...], preferred_element_type=jnp.float32) \
            + b2_ref[...]

        b = batch_ref[...]
        gid = lax.broadcasted_iota(jnp.int32, (BLK, G), 1)
        maskT = (jnp.broadcast_to(b, (BLK, G)) == gid).astype(jnp.float32)
        pooled_s[...] += lax.dot_general(
            maskT, h, (((0,), (0,)), ((), ())),
            preferred_element_type=jnp.float32)
        counts_s[...] += lax.dot_general(
            maskT, jnp.ones((BLK, D), jnp.float32), (((0,), (0,)), ((), ())),
            preferred_element_type=jnp.float32)

        @pl.when(i == TC_GRID - 1)
        def _():
            pm = pooled_s[...] / jnp.maximum(counts_s[...], 1.0)
            res = jnp.dot(pm, fcw_ref[...],
                          preferred_element_type=jnp.float32)
            o_ref[...] = jnp.broadcast_to(res, (G, D)) + fcb_ref[0]

    row_spec = pl.BlockSpec((BLK, D), lambda i: (i, 0))
    full_spec = pl.BlockSpec((D, D), lambda i: (0, 0))
    bias_spec = pl.BlockSpec((1, D), lambda i: (0, 0))
    return pl.pallas_call(
        body,
        grid=(TC_GRID,),
        in_specs=[
            pl.BlockSpec(memory_space=pltpu.SMEM),
            pl.BlockSpec(memory_space=pltpu.SMEM),
            row_spec, row_spec, row_spec,
            full_spec, bias_spec, full_spec, bias_spec,
            pl.BlockSpec((BLK, 1), lambda i: (i, 0)),
            pl.BlockSpec((D, 1), lambda i: (0, 0)),
        ],
        out_specs=pl.BlockSpec((G, D), lambda i: (0, 0)),
        out_shape=jax.ShapeDtypeStruct((G, D), jnp.float32),
        scratch_shapes=[
            pltpu.VMEM((G, D), jnp.float32),
            pltpu.VMEM((G, D), jnp.float32),
        ],
    )(eps.reshape(1), fcb, x, a0, a1, W1, b1.reshape(1, D), W2,
      b2.reshape(1, D), batch2d, fcW)


def kernel(x, edge_index, batch, W1_0, b1_0, W2_0, b2_0, eps_0, W1_1, b1_1,
           W2_1, b2_1, eps_1, W1_2, b1_2, W2_2, b2_2, eps_2, fcW, fcb):
    src = edge_index[0]
    dst = edge_index[1]
    pad = E_PAD - E
    src3 = jnp.concatenate([src, jnp.zeros((pad,), jnp.int32)]) \
        .reshape(NW, K, CHUNK)
    dst3 = jnp.concatenate([dst, jnp.full((pad,), N, jnp.int32)]) \
        .reshape(NW, K, CHUNK)
    x_pad = jnp.concatenate([x, jnp.zeros((N_PAD - N, D), jnp.float32)])
    batch2d = jnp.concatenate([batch, jnp.full((N_PAD - N,), G, jnp.int32)]) \
        .reshape(N_PAD, 1)
    zrows = jnp.zeros((ZR, D), jnp.float32)

    params = [(W1_0, b1_0, W2_0, b2_0, eps_0),
              (W1_1, b1_1, W2_1, b2_1, eps_1),
              (W1_2, b1_2, W2_2, b2_2, eps_2)]

    h = x_pad
    for i, (W1, b1, W2, b2, eps) in enumerate(params):
        agg = _segment_sum_sc(h, src3, dst3, zrows)
        if i < 2:
            h = _mlp_tc(h, agg[0], agg[1], W1, b1, W2, b2, eps,
                        outer_relu=True)
        else:
            out2d = _mlp_pool_tc(h, agg[0], agg[1], W1, b1, W2, b2, eps,
                                 batch2d, fcW, fcb)
    return out2d[:, 0]
